# baseline (speedup 1.0000x reference)
import functools
import math

import jax
import jax.numpy as jnp
from jax import lax
from jax.experimental import pallas as pl
from jax.experimental.pallas import tpu as pltpu
from jax.experimental.pallas import tpu_sc as plsc

N = 10000
E = 320000
D = 128
ED = 16
DCUT = 5.0

EDGE_BLK = 2000
NODE_BLK = 1000

NC = 2
NS = 16
CHUNK = 128
NCHUNK = E // CHUNK
ROWS_PER_SUB = 624
TAIL = N - NS * ROWS_PER_SUB


def _sc_mesh():
    return plsc.VectorSubcoreMesh(core_axis_name="c", subcore_axis_name="s")


def _sc_gather(table, idx2d, g0, nch):
    NW = NC * NS
    KMAX = (nch + NW - 1) // NW
    if KMAX % 2:
        KMAX += 1

    @functools.partial(
        pl.kernel,
        mesh=_sc_mesh(),
        out_type=jax.ShapeDtypeStruct((nch * CHUNK, D), jnp.float32),
        scratch_types=[
            pltpu.VMEM((2, CHUNK), jnp.int32),
            pltpu.VMEM((2, CHUNK, D), jnp.float32),
            pltpu.SemaphoreType.DMA,
            pltpu.SemaphoreType.DMA,
            pltpu.SemaphoreType.DMA,
            pltpu.SemaphoreType.DMA,
            pltpu.SemaphoreType.DMA,
            pltpu.SemaphoreType.DMA,
        ],
    )
    def k(table_hbm, idx_hbm, out_hbm, idx_v, rows_v, si0, si1, sg0, sg1,
          ss0, ss1):
        si = (si0, si1)
        sg = (sg0, sg1)
        ss = (ss0, ss1)
        wid = lax.axis_index("s") * NC + lax.axis_index("c")

        def idx_cp(lg, j):
            return pltpu.make_async_copy(idx_hbm.at[g0 + lg], idx_v.at[j],
                                         si[j])

        def gat_cp(j):
            return pltpu.make_async_copy(table_hbm.at[idx_v.at[j]],
                                         rows_v.at[j], sg[j])

        def st_cp(lg, j):
            return pltpu.make_async_copy(
                rows_v.at[j], out_hbm.at[pl.ds(lg * CHUNK, CHUNK)], ss[j])

        for j in range(2):
            idx_cp(wid + j * NW, j).start()
        for j in range(2):
            idx_cp(wid + j * NW, j).wait()
            gat_cp(j).start()

        @pl.loop(0, KMAX, step=2)
        def _(kk):
            for j in range(2):
                lg = wid + (kk + j) * NW
                lgn = wid + (kk + j + 2) * NW

                @pl.when(lg < nch)
                def _():
                    gat_cp(j).wait()

                    @pl.when(lgn < nch)
                    def _():
                        idx_cp(lgn, j).start()

                    st_cp(lg, j).start()

                    @pl.when(lgn < nch)
                    def _():
                        idx_cp(lgn, j).wait()
                        st_cp(lg, j).wait()
                        gat_cp(j).start()

        for j in range(2):
            st_cp(0, j).wait()

    return k(table, idx2d)


def _sc_scatter_add(msg, dst2d, zrows, g0, nch):
    per_core = nch // NC

    @functools.partial(
        pl.kernel,
        mesh=_sc_mesh(),
        out_type=jax.ShapeDtypeStruct((NC, N, D), jnp.float32),
        scratch_types=[
            pltpu.VMEM((2, CHUNK), jnp.int32),
            pltpu.VMEM((2, CHUNK, D), jnp.float32),
            pltpu.VMEM_SHARED((N, D), jnp.float32),
            pltpu.SemaphoreType.DMA,
            pltpu.SemaphoreType.DMA,
            pltpu.SemaphoreType.DMA,
            pltpu.SemaphoreType.DMA,
            pltpu.SemaphoreType.DMA,
            pltpu.SemaphoreType.DMA,
        ],
    )
    def k(msg_hbm, dst_hbm, zero_hbm, out_hbm, idx_v, rows_v, acc_sh,
          si0, si1, sr0, sr1, sa0, sa1):
        si = (si0, si1)
        sr = (sr0, sr1)
        sa = (sa0, sa1)
        cid = lax.axis_index("c")
        sid = lax.axis_index("s")
        pltpu.sync_copy(zero_hbm, acc_sh.at[pl.ds(sid * ROWS_PER_SUB,
                                                  ROWS_PER_SUB)])

        @pl.when(sid == 0)
        def _():
            pltpu.sync_copy(zero_hbm.at[pl.ds(0, TAIL)],
                            acc_sh.at[pl.ds(NS * ROWS_PER_SUB, TAIL)])

        plsc.subcore_barrier()

        KMAX = (per_core + NS - 1) // NS
        if KMAX % 2:
            KMAX += 1
        base = cid * per_core + sid
        limit = (cid + 1) * per_core

        def idx_cp(lg, j):
            return pltpu.make_async_copy(dst_hbm.at[g0 + lg], idx_v.at[j],
                                         si[j])

        def row_cp(lg, j):
            return pltpu.make_async_copy(
                msg_hbm.at[pl.ds(lg * CHUNK, CHUNK)], rows_v.at[j], sr[j])

        def add_cp(j):
            return pltpu.make_async_copy(rows_v.at[j],
                                         acc_sh.at[idx_v.at[j]], sa[j])

        for j in range(2):
            idx_cp(base + j * NS, j).start()
            row_cp(base + j * NS, j).start()

        @pl.loop(0, KMAX, step=2)
        def _(kk):
            for j in range(2):
                lg = base + (kk + j) * NS
                lgn = base + (kk + j + 2) * NS

                @pl.when(lg < limit)
                def _():
                    idx_cp(lg, j).wait()
                    row_cp(lg, j).wait()
                    pltpu.async_copy(rows_v.at[j], acc_sh.at[idx_v.at[j]],
                                     sa[j], add=True)

                    @pl.when(lgn < limit)
                    def _():
                        add_cp(j).wait()
                        idx_cp(lgn, j).start()
                        row_cp(lgn, j).start()

        for j in range(2):
            add_cp(j).wait()

        plsc.subcore_barrier()
        pltpu.sync_copy(
            acc_sh.at[pl.ds(sid * ROWS_PER_SUB, ROWS_PER_SUB)],
            out_hbm.at[cid].at[pl.ds(sid * ROWS_PER_SUB, ROWS_PER_SUB)])

        @pl.when(sid == 0)
        def _():
            pltpu.sync_copy(
                acc_sh.at[pl.ds(NS * ROWS_PER_SUB, TAIL)],
                out_hbm.at[cid].at[pl.ds(NS * ROWS_PER_SUB, TAIL)])

    return k(msg, dst2d, zrows)


def _edge_scalar_body(ew_ref, cut_ref, t_ref):
    d = ew_ref[...]
    dc = jnp.minimum(d, DCUT)
    cut_ref[...] = 0.5 * (jnp.cos(dc * (math.pi / DCUT)) + 1.0)
    t_ref[...] = jnp.exp(-d)


def _edge_scalars(ew_g):
    return pl.pallas_call(
        _edge_scalar_body,
        grid=(1,),
        in_specs=[pl.BlockSpec((NCHUNK, CHUNK), lambda i: (0, 0))],
        out_specs=[pl.BlockSpec((NCHUNK, CHUNK), lambda i: (0, 0))] * 2,
        out_shape=[jax.ShapeDtypeStruct((NCHUNK, CHUNK), jnp.float32)] * 2,
    )(ew_g)


def _edge_msg_body(comb_ref, xj_ref, Wd_ref, bd_ref, Wl_ref,
                   bl_ref, out_ref):
    start = math.exp(-DCUT)
    beta = (2.0 / D * (1.0 - start)) ** (-2)
    means = start + lax.broadcasted_iota(jnp.int32, (1, D), 1).astype(
        jnp.float32) * ((1.0 - start) / (D - 1))
    comb = comb_ref[...]
    ea = comb[:, :ED]
    cut = comb[:, ED:ED + 1]
    t = comb[:, ED + 1:ED + 2]
    feat = cut * jnp.exp(-beta * (t - means) ** 2)
    ss = jnp.dot(feat, Wd_ref[...], preferred_element_type=jnp.float32)
    ss = ss + bd_ref[...]
    scale = ss[:, :D]
    shift = ss[:, D:]
    eap = jnp.dot(ea, Wl_ref[...],
                  preferred_element_type=jnp.float32) + bl_ref[...]
    out_ref[...] = jnp.maximum((xj_ref[...] + eap) * scale + shift, 0.0)


def _edge_messages(comb, xj, Wd, bd, Wlc, bl, blk0, nblk):
    nrows = nblk * EDGE_BLK
    return pl.pallas_call(
        _edge_msg_body,
        grid=(nblk,),
        in_specs=[
            pl.BlockSpec((EDGE_BLK, D), lambda i: (i + blk0, 0)),
            pl.BlockSpec((EDGE_BLK, D), lambda i: (i, 0)),
            pl.BlockSpec((D, 2 * D), lambda i: (0, 0)),
            pl.BlockSpec((1, 2 * D), lambda i: (0, 0)),
            pl.BlockSpec((ED, D), lambda i: (0, 0)),
            pl.BlockSpec((1, D), lambda i: (0, 0)),
        ],
        out_specs=pl.BlockSpec((EDGE_BLK, D), lambda i: (i, 0)),
        out_shape=jax.ShapeDtypeStruct((nrows, D), jnp.float32),
    )(comb, xj, Wd, bd, Wlc, bl)


def _node_mlp_body(p00_ref, p01_ref, p10_ref, p11_ref, x_ref, W1_ref, b1_ref,
                   W2_ref, b2_ref, out_ref, *, apply_relu):
    out = (p00_ref[0] + p01_ref[0] + p10_ref[0] + p11_ref[0] + x_ref[...])
    h = jnp.maximum(
        jnp.dot(out, W1_ref[...], preferred_element_type=jnp.float32)
        + b1_ref[...], 0.0)
    h = jnp.dot(h, W2_ref[...], preferred_element_type=jnp.float32) + b2_ref[...]
    if apply_relu:
        h = jnp.maximum(h, 0.0)
    out_ref[...] = h


def _node_mlp(parts0, parts1, x, W1, b1, W2, b2, apply_relu):
    grid = (N // NODE_BLK,)
    return pl.pallas_call(
        functools.partial(_node_mlp_body, apply_relu=apply_relu),
        grid=grid,
        in_specs=[
            pl.BlockSpec((1, NODE_BLK, D), lambda i: (0, i, 0)),
            pl.BlockSpec((1, NODE_BLK, D), lambda i: (1, i, 0)),
            pl.BlockSpec((1, NODE_BLK, D), lambda i: (0, i, 0)),
            pl.BlockSpec((1, NODE_BLK, D), lambda i: (1, i, 0)),
            pl.BlockSpec((NODE_BLK, D), lambda i: (i, 0)),
            pl.BlockSpec((D, D), lambda i: (0, 0)),
            pl.BlockSpec((1, D), lambda i: (0, 0)),
            pl.BlockSpec((D, D), lambda i: (0, 0)),
            pl.BlockSpec((1, D), lambda i: (0, 0)),
        ],
        out_specs=pl.BlockSpec((NODE_BLK, D), lambda i: (i, 0)),
        out_shape=jax.ShapeDtypeStruct((N, D), jnp.float32),
    )(parts0, parts0, parts1, parts1, x, W1, b1, W2, b2)


def kernel(x, edge_weight, edge_attr, We0, Wd0, bd0, Wl0, bl0, W1_0, b1_0,
           W2_0, b2_0, Wd1, bd1, Wl1, bl1, W1_1, b1_1, W2_1, b2_1, edge_index):
    ei2d = edge_index.reshape(2, NCHUNK, CHUNK)
    src2d = ei2d[0]
    dst2d = ei2d[1]
    cut_g, t_g = _edge_scalars(edge_weight.reshape(NCHUNK, CHUNK))
    comb = jnp.concatenate(
        [edge_attr, cut_g.reshape(E, 1), t_g.reshape(E, 1),
         jnp.zeros((E, D - ED - 2), jnp.float32)], axis=1)
    Wlc0 = We0 @ Wl0
    Wlc1 = We0 @ Wl1
    zrows = jnp.zeros((ROWS_PER_SUB, D), jnp.float32)

    HCH = NCHUNK // 2
    HBLK = (HCH * CHUNK) // EDGE_BLK

    def layer(h, Wd, bd, Wlc, bl, W1, b1, W2, b2, apply_relu):
        xj0 = _sc_gather(h, src2d, 0, HCH)
        xj1 = _sc_gather(h, src2d, HCH, HCH)
        msg0 = _edge_messages(comb, xj0, Wd, bd[None, :], Wlc, bl[None, :],
                              0, HBLK)
        msg1 = _edge_messages(comb, xj1, Wd, bd[None, :], Wlc, bl[None, :],
                              HBLK, HBLK)
        parts0 = _sc_scatter_add(msg0, dst2d, zrows, 0, HCH)
        parts1 = _sc_scatter_add(msg1, dst2d, zrows, HCH, HCH)
        return _node_mlp(parts0, parts1, h, W1, b1[None, :], W2, b2[None, :],
                         apply_relu)

    h = layer(x, Wd0, bd0, Wlc0, bl0, W1_0, b1_0, W2_0, b2_0, True)
    h = layer(h, Wd1, bd1, Wlc1, bl1, W1_1, b1_1, W2_1, b2_1, False)
    return h


# speedup vs baseline: 1.3541x; 1.3541x over previous
import functools
import math

import jax
import jax.numpy as jnp
from jax import lax
from jax.experimental import pallas as pl
from jax.experimental.pallas import tpu as pltpu
from jax.experimental.pallas import tpu_sc as plsc

N = 10000
E = 320000
D = 128
ED = 16
DCUT = 5.0

EDGE_BLK = 4000
NODE_BLK = 1000

NC = 2
NS = 16
CHUNK = 128
NCHUNK = E // CHUNK
ROWS_PER_SUB = 624
TAIL = N - NS * ROWS_PER_SUB


def _sc_mesh():
    return plsc.VectorSubcoreMesh(core_axis_name="c", subcore_axis_name="s")


def _sc_gather(table, idx2d, g0, nch):
    NW = NC * NS
    KMAX = (nch + NW - 1) // NW
    if KMAX % 2:
        KMAX += 1

    @functools.partial(
        pl.kernel,
        mesh=_sc_mesh(),
        out_type=jax.ShapeDtypeStruct((nch * CHUNK, D), jnp.float32),
        scratch_types=[
            pltpu.VMEM((2, CHUNK), jnp.int32),
            pltpu.VMEM((2, CHUNK, D), jnp.float32),
            pltpu.SemaphoreType.DMA,
            pltpu.SemaphoreType.DMA,
            pltpu.SemaphoreType.DMA,
            pltpu.SemaphoreType.DMA,
            pltpu.SemaphoreType.DMA,
            pltpu.SemaphoreType.DMA,
        ],
    )
    def k(table_hbm, idx_hbm, out_hbm, idx_v, rows_v, si0, si1, sg0, sg1,
          ss0, ss1):
        si = (si0, si1)
        sg = (sg0, sg1)
        ss = (ss0, ss1)
        wid = lax.axis_index("s") * NC + lax.axis_index("c")

        def idx_cp(lg, j):
            return pltpu.make_async_copy(idx_hbm.at[g0 + lg], idx_v.at[j],
                                         si[j])

        def gat_cp(j):
            return pltpu.make_async_copy(table_hbm.at[idx_v.at[j]],
                                         rows_v.at[j], sg[j])

        def st_cp(lg, j):
            return pltpu.make_async_copy(
                rows_v.at[j], out_hbm.at[pl.ds(lg * CHUNK, CHUNK)], ss[j])

        for j in range(2):
            idx_cp(wid + j * NW, j).start()
        for j in range(2):
            idx_cp(wid + j * NW, j).wait()
            gat_cp(j).start()

        @pl.loop(0, KMAX, step=2)
        def _(kk):
            for j in range(2):
                lg = wid + (kk + j) * NW
                lgn = wid + (kk + j + 2) * NW

                @pl.when(lg < nch)
                def _():
                    gat_cp(j).wait()

                    @pl.when(lgn < nch)
                    def _():
                        idx_cp(lgn, j).start()

                    st_cp(lg, j).start()

                    @pl.when(lgn < nch)
                    def _():
                        idx_cp(lgn, j).wait()
                        st_cp(lg, j).wait()
                        gat_cp(j).start()

        for j in range(2):
            st_cp(0, j).wait()

    return k(table, idx2d)


def _sc_scatter_add(msg, dst2d, zrows, g0, nch):
    per_core = nch // NC

    @functools.partial(
        pl.kernel,
        mesh=_sc_mesh(),
        out_type=jax.ShapeDtypeStruct((NC, N, D), jnp.float32),
        scratch_types=[
            pltpu.VMEM((2, CHUNK), jnp.int32),
            pltpu.VMEM((2, CHUNK, D), jnp.float32),
            pltpu.VMEM_SHARED((N, D), jnp.float32),
            pltpu.SemaphoreType.DMA,
            pltpu.SemaphoreType.DMA,
            pltpu.SemaphoreType.DMA,
            pltpu.SemaphoreType.DMA,
            pltpu.SemaphoreType.DMA,
            pltpu.SemaphoreType.DMA,
        ],
    )
    def k(msg_hbm, dst_hbm, zero_hbm, out_hbm, idx_v, rows_v, acc_sh,
          si0, si1, sr0, sr1, sa0, sa1):
        si = (si0, si1)
        sr = (sr0, sr1)
        sa = (sa0, sa1)
        cid = lax.axis_index("c")
        sid = lax.axis_index("s")
        pltpu.sync_copy(zero_hbm, acc_sh.at[pl.ds(sid * ROWS_PER_SUB,
                                                  ROWS_PER_SUB)])

        @pl.when(sid == 0)
        def _():
            pltpu.sync_copy(zero_hbm.at[pl.ds(0, TAIL)],
                            acc_sh.at[pl.ds(NS * ROWS_PER_SUB, TAIL)])

        plsc.subcore_barrier()

        KMAX = (per_core + NS - 1) // NS
        if KMAX % 2:
            KMAX += 1
        base = cid * per_core + sid
        limit = (cid + 1) * per_core

        def idx_cp(lg, j):
            return pltpu.make_async_copy(dst_hbm.at[g0 + lg], idx_v.at[j],
                                         si[j])

        def row_cp(lg, j):
            return pltpu.make_async_copy(
                msg_hbm.at[pl.ds(lg * CHUNK, CHUNK)], rows_v.at[j], sr[j])

        def add_cp(j):
            return pltpu.make_async_copy(rows_v.at[j],
                                         acc_sh.at[idx_v.at[j]], sa[j])

        for j in range(2):
            idx_cp(base + j * NS, j).start()
            row_cp(base + j * NS, j).start()

        @pl.loop(0, KMAX, step=2)
        def _(kk):
            for j in range(2):
                lg = base + (kk + j) * NS
                lgn = base + (kk + j + 2) * NS

                @pl.when(lg < limit)
                def _():
                    idx_cp(lg, j).wait()
                    row_cp(lg, j).wait()
                    pltpu.async_copy(rows_v.at[j], acc_sh.at[idx_v.at[j]],
                                     sa[j], add=True)

                    @pl.when(lgn < limit)
                    def _():
                        add_cp(j).wait()
                        idx_cp(lgn, j).start()
                        row_cp(lgn, j).start()

        for j in range(2):
            add_cp(j).wait()

        plsc.subcore_barrier()
        pltpu.sync_copy(
            acc_sh.at[pl.ds(sid * ROWS_PER_SUB, ROWS_PER_SUB)],
            out_hbm.at[cid].at[pl.ds(sid * ROWS_PER_SUB, ROWS_PER_SUB)])

        @pl.when(sid == 0)
        def _():
            pltpu.sync_copy(
                acc_sh.at[pl.ds(NS * ROWS_PER_SUB, TAIL)],
                out_hbm.at[cid].at[pl.ds(NS * ROWS_PER_SUB, TAIL)])

    return k(msg, dst2d, zrows)


def _edge_scalar_body(ew_ref, cut_ref, t_ref):
    d = ew_ref[...]
    dc = jnp.minimum(d, DCUT)
    cut_ref[...] = 0.5 * (jnp.cos(dc * (math.pi / DCUT)) + 1.0)
    t_ref[...] = jnp.exp(-d)


def _edge_scalars(ew_g):
    return pl.pallas_call(
        _edge_scalar_body,
        grid=(1,),
        in_specs=[pl.BlockSpec((NCHUNK, CHUNK), lambda i: (0, 0))],
        out_specs=[pl.BlockSpec((NCHUNK, CHUNK), lambda i: (0, 0))] * 2,
        out_shape=[jax.ShapeDtypeStruct((NCHUNK, CHUNK), jnp.float32)] * 2,
    )(ew_g)


def _edge_msg_body(comb_ref, xj_ref, Wd_ref, bd_ref, Wl_ref,
                   bl_ref, out_ref):
    start = math.exp(-DCUT)
    beta = (2.0 / D * (1.0 - start)) ** (-2)
    means = start + lax.broadcasted_iota(jnp.int32, (1, D), 1).astype(
        jnp.float32) * ((1.0 - start) / (D - 1))
    comb = comb_ref[...]
    ea = comb[:, :ED]
    cut = comb[:, ED:ED + 1]
    t = comb[:, ED + 1:ED + 2]
    feat = cut * jnp.exp(-beta * (t - means) ** 2)
    ss = jnp.dot(feat, Wd_ref[...], preferred_element_type=jnp.float32)
    ss = ss + bd_ref[...]
    scale = ss[:, :D]
    shift = ss[:, D:]
    eap = jnp.dot(ea, Wl_ref[...],
                  preferred_element_type=jnp.float32) + bl_ref[...]
    out_ref[...] = jnp.maximum((xj_ref[...] + eap) * scale + shift, 0.0)


def _edge_messages(comb, xj, Wd, bd, Wlc, bl, blk0, nblk):
    nrows = nblk * EDGE_BLK
    return pl.pallas_call(
        _edge_msg_body,
        grid=(nblk,),
        in_specs=[
            pl.BlockSpec((EDGE_BLK, ED + 2), lambda i: (i + blk0, 0)),
            pl.BlockSpec((EDGE_BLK, D), lambda i: (i, 0)),
            pl.BlockSpec((D, 2 * D), lambda i: (0, 0)),
            pl.BlockSpec((1, 2 * D), lambda i: (0, 0)),
            pl.BlockSpec((ED, D), lambda i: (0, 0)),
            pl.BlockSpec((1, D), lambda i: (0, 0)),
        ],
        out_specs=pl.BlockSpec((EDGE_BLK, D), lambda i: (i, 0)),
        out_shape=jax.ShapeDtypeStruct((nrows, D), jnp.float32),
    )(comb, xj, Wd, bd, Wlc, bl)


def _node_mlp_body(p00_ref, p01_ref, p10_ref, p11_ref, x_ref, W1_ref, b1_ref,
                   W2_ref, b2_ref, out_ref, *, apply_relu):
    out = (p00_ref[0] + p01_ref[0] + p10_ref[0] + p11_ref[0] + x_ref[...])
    h = jnp.maximum(
        jnp.dot(out, W1_ref[...], preferred_element_type=jnp.float32)
        + b1_ref[...], 0.0)
    h = jnp.dot(h, W2_ref[...], preferred_element_type=jnp.float32) + b2_ref[...]
    if apply_relu:
        h = jnp.maximum(h, 0.0)
    out_ref[...] = h


def _node_mlp(parts0, parts1, x, W1, b1, W2, b2, apply_relu):
    grid = (N // NODE_BLK,)
    return pl.pallas_call(
        functools.partial(_node_mlp_body, apply_relu=apply_relu),
        grid=grid,
        in_specs=[
            pl.BlockSpec((1, NODE_BLK, D), lambda i: (0, i, 0)),
            pl.BlockSpec((1, NODE_BLK, D), lambda i: (1, i, 0)),
            pl.BlockSpec((1, NODE_BLK, D), lambda i: (0, i, 0)),
            pl.BlockSpec((1, NODE_BLK, D), lambda i: (1, i, 0)),
            pl.BlockSpec((NODE_BLK, D), lambda i: (i, 0)),
            pl.BlockSpec((D, D), lambda i: (0, 0)),
            pl.BlockSpec((1, D), lambda i: (0, 0)),
            pl.BlockSpec((D, D), lambda i: (0, 0)),
            pl.BlockSpec((1, D), lambda i: (0, 0)),
        ],
        out_specs=pl.BlockSpec((NODE_BLK, D), lambda i: (i, 0)),
        out_shape=jax.ShapeDtypeStruct((N, D), jnp.float32),
    )(parts0, parts0, parts1, parts1, x, W1, b1, W2, b2)


def kernel(x, edge_weight, edge_attr, We0, Wd0, bd0, Wl0, bl0, W1_0, b1_0,
           W2_0, b2_0, Wd1, bd1, Wl1, bl1, W1_1, b1_1, W2_1, b2_1, edge_index):
    ei2d = edge_index.reshape(2, NCHUNK, CHUNK)
    src2d = ei2d[0]
    dst2d = ei2d[1]
    cut_g, t_g = _edge_scalars(edge_weight.reshape(NCHUNK, CHUNK))
    comb = jnp.concatenate(
        [edge_attr, cut_g.reshape(E, 1), t_g.reshape(E, 1)], axis=1)
    Wlc0 = We0 @ Wl0
    Wlc1 = We0 @ Wl1
    zrows = jnp.zeros((ROWS_PER_SUB, D), jnp.float32)

    HCH = NCHUNK // 2
    HBLK = (HCH * CHUNK) // EDGE_BLK

    def layer(h, Wd, bd, Wlc, bl, W1, b1, W2, b2, apply_relu):
        xj0 = _sc_gather(h, src2d, 0, HCH)
        xj1 = _sc_gather(h, src2d, HCH, HCH)
        msg0 = _edge_messages(comb, xj0, Wd, bd[None, :], Wlc, bl[None, :],
                              0, HBLK)
        msg1 = _edge_messages(comb, xj1, Wd, bd[None, :], Wlc, bl[None, :],
                              HBLK, HBLK)
        parts0 = _sc_scatter_add(msg0, dst2d, zrows, 0, HCH)
        parts1 = _sc_scatter_add(msg1, dst2d, zrows, HCH, HCH)
        return _node_mlp(parts0, parts1, h, W1, b1[None, :], W2, b2[None, :],
                         apply_relu)

    h = layer(x, Wd0, bd0, Wlc0, bl0, W1_0, b1_0, W2_0, b2_0, True)
    h = layer(h, Wd1, bd1, Wlc1, bl1, W1_1, b1_1, W2_1, b2_1, False)
    return h


# speedup vs baseline: 1.3889x; 1.0257x over previous
import functools
import math

import jax
import jax.numpy as jnp
from jax import lax
from jax.experimental import pallas as pl
from jax.experimental.pallas import tpu as pltpu
from jax.experimental.pallas import tpu_sc as plsc

N = 10000
E = 320000
D = 128
ED = 16
DCUT = 5.0

EDGE_BLK = 8000
NODE_BLK = 1000

NC = 2
NS = 16
CHUNK = 128
NCHUNK = E // CHUNK
ROWS_PER_SUB = 624
TAIL = N - NS * ROWS_PER_SUB


def _sc_mesh():
    return plsc.VectorSubcoreMesh(core_axis_name="c", subcore_axis_name="s")


def _sc_gather(table, idx2d, g0, nch):
    NW = NC * NS
    KMAX = (nch + NW - 1) // NW
    if KMAX % 2:
        KMAX += 1

    @functools.partial(
        pl.kernel,
        mesh=_sc_mesh(),
        out_type=jax.ShapeDtypeStruct((nch * CHUNK, D), jnp.float32),
        scratch_types=[
            pltpu.VMEM((2, CHUNK), jnp.int32),
            pltpu.VMEM((2, CHUNK, D), jnp.float32),
            pltpu.SemaphoreType.DMA,
            pltpu.SemaphoreType.DMA,
            pltpu.SemaphoreType.DMA,
            pltpu.SemaphoreType.DMA,
            pltpu.SemaphoreType.DMA,
            pltpu.SemaphoreType.DMA,
        ],
    )
    def k(table_hbm, idx_hbm, out_hbm, idx_v, rows_v, si0, si1, sg0, sg1,
          ss0, ss1):
        si = (si0, si1)
        sg = (sg0, sg1)
        ss = (ss0, ss1)
        wid = lax.axis_index("s") * NC + lax.axis_index("c")

        def idx_cp(lg, j):
            return pltpu.make_async_copy(idx_hbm.at[g0 + lg], idx_v.at[j],
                                         si[j])

        def gat_cp(j):
            return pltpu.make_async_copy(table_hbm.at[idx_v.at[j]],
                                         rows_v.at[j], sg[j])

        def st_cp(lg, j):
            return pltpu.make_async_copy(
                rows_v.at[j], out_hbm.at[pl.ds(lg * CHUNK, CHUNK)], ss[j])

        for j in range(2):
            idx_cp(wid + j * NW, j).start()
        for j in range(2):
            idx_cp(wid + j * NW, j).wait()
            gat_cp(j).start()

        @pl.loop(0, KMAX, step=2)
        def _(kk):
            for j in range(2):
                lg = wid + (kk + j) * NW
                lgn = wid + (kk + j + 2) * NW

                @pl.when(lg < nch)
                def _():
                    gat_cp(j).wait()

                    @pl.when(lgn < nch)
                    def _():
                        idx_cp(lgn, j).start()

                    st_cp(lg, j).start()

                    @pl.when(lgn < nch)
                    def _():
                        idx_cp(lgn, j).wait()
                        st_cp(lg, j).wait()
                        gat_cp(j).start()

        for j in range(2):
            st_cp(0, j).wait()

    return k(table, idx2d)


def _sc_scatter_add(msg, dst2d, zrows, g0, nch):
    per_core = nch // NC

    @functools.partial(
        pl.kernel,
        mesh=_sc_mesh(),
        out_type=jax.ShapeDtypeStruct((NC, N, D), jnp.float32),
        scratch_types=[
            pltpu.VMEM((2, CHUNK), jnp.int32),
            pltpu.VMEM((2, CHUNK, D), jnp.float32),
            pltpu.VMEM_SHARED((N, D), jnp.float32),
            pltpu.SemaphoreType.DMA,
            pltpu.SemaphoreType.DMA,
            pltpu.SemaphoreType.DMA,
            pltpu.SemaphoreType.DMA,
            pltpu.SemaphoreType.DMA,
            pltpu.SemaphoreType.DMA,
        ],
    )
    def k(msg_hbm, dst_hbm, zero_hbm, out_hbm, idx_v, rows_v, acc_sh,
          si0, si1, sr0, sr1, sa0, sa1):
        si = (si0, si1)
        sr = (sr0, sr1)
        sa = (sa0, sa1)
        cid = lax.axis_index("c")
        sid = lax.axis_index("s")
        pltpu.sync_copy(zero_hbm, acc_sh.at[pl.ds(sid * ROWS_PER_SUB,
                                                  ROWS_PER_SUB)])

        @pl.when(sid == 0)
        def _():
            pltpu.sync_copy(zero_hbm.at[pl.ds(0, TAIL)],
                            acc_sh.at[pl.ds(NS * ROWS_PER_SUB, TAIL)])

        plsc.subcore_barrier()

        KMAX = (per_core + NS - 1) // NS
        if KMAX % 2:
            KMAX += 1
        base = cid * per_core + sid
        limit = (cid + 1) * per_core

        def idx_cp(lg, j):
            return pltpu.make_async_copy(dst_hbm.at[g0 + lg], idx_v.at[j],
                                         si[j])

        def row_cp(lg, j):
            return pltpu.make_async_copy(
                msg_hbm.at[pl.ds(lg * CHUNK, CHUNK)], rows_v.at[j], sr[j])

        def add_cp(j):
            return pltpu.make_async_copy(rows_v.at[j],
                                         acc_sh.at[idx_v.at[j]], sa[j])

        for j in range(2):
            idx_cp(base + j * NS, j).start()
            row_cp(base + j * NS, j).start()

        @pl.loop(0, KMAX, step=2)
        def _(kk):
            for j in range(2):
                lg = base + (kk + j) * NS
                lgn = base + (kk + j + 2) * NS

                @pl.when(lg < limit)
                def _():
                    idx_cp(lg, j).wait()
                    row_cp(lg, j).wait()
                    pltpu.async_copy(rows_v.at[j], acc_sh.at[idx_v.at[j]],
                                     sa[j], add=True)

                    @pl.when(lgn < limit)
                    def _():
                        add_cp(j).wait()
                        idx_cp(lgn, j).start()
                        row_cp(lgn, j).start()

        for j in range(2):
            add_cp(j).wait()

        plsc.subcore_barrier()
        pltpu.sync_copy(
            acc_sh.at[pl.ds(sid * ROWS_PER_SUB, ROWS_PER_SUB)],
            out_hbm.at[cid].at[pl.ds(sid * ROWS_PER_SUB, ROWS_PER_SUB)])

        @pl.when(sid == 0)
        def _():
            pltpu.sync_copy(
                acc_sh.at[pl.ds(NS * ROWS_PER_SUB, TAIL)],
                out_hbm.at[cid].at[pl.ds(NS * ROWS_PER_SUB, TAIL)])

    return k(msg, dst2d, zrows)


def _edge_scalar_body(ew_ref, cut_ref, t_ref):
    d = ew_ref[...]
    dc = jnp.minimum(d, DCUT)
    cut_ref[...] = 0.5 * (jnp.cos(dc * (math.pi / DCUT)) + 1.0)
    t_ref[...] = jnp.exp(-d)


def _edge_scalars(ew_g):
    return pl.pallas_call(
        _edge_scalar_body,
        grid=(1,),
        in_specs=[pl.BlockSpec((NCHUNK, CHUNK), lambda i: (0, 0))],
        out_specs=[pl.BlockSpec((NCHUNK, CHUNK), lambda i: (0, 0))] * 2,
        out_shape=[jax.ShapeDtypeStruct((NCHUNK, CHUNK), jnp.float32)] * 2,
    )(ew_g)


def _edge_msg_body(comb_ref, xj_ref, Wd_ref, bd_ref, Wl_ref,
                   bl_ref, out_ref):
    start = math.exp(-DCUT)
    beta = (2.0 / D * (1.0 - start)) ** (-2)
    means = start + lax.broadcasted_iota(jnp.int32, (1, D), 1).astype(
        jnp.float32) * ((1.0 - start) / (D - 1))
    comb = comb_ref[...]
    ea = comb[:, :ED]
    cut = comb[:, ED:ED + 1]
    t = comb[:, ED + 1:ED + 2]
    feat = cut * jnp.exp(-beta * (t - means) ** 2)
    ss = jnp.dot(feat, Wd_ref[...], preferred_element_type=jnp.float32)
    ss = ss + bd_ref[...]
    scale = ss[:, :D]
    shift = ss[:, D:]
    eap = jnp.dot(ea, Wl_ref[...],
                  preferred_element_type=jnp.float32) + bl_ref[...]
    out_ref[...] = jnp.maximum((xj_ref[...] + eap) * scale + shift, 0.0)


def _edge_messages(comb, xj, Wd, bd, Wlc, bl, blk0, nblk):
    nrows = nblk * EDGE_BLK
    return pl.pallas_call(
        _edge_msg_body,
        grid=(nblk,),
        in_specs=[
            pl.BlockSpec((EDGE_BLK, ED + 2), lambda i: (i + blk0, 0)),
            pl.BlockSpec((EDGE_BLK, D), lambda i: (i, 0)),
            pl.BlockSpec((D, 2 * D), lambda i: (0, 0)),
            pl.BlockSpec((1, 2 * D), lambda i: (0, 0)),
            pl.BlockSpec((ED, D), lambda i: (0, 0)),
            pl.BlockSpec((1, D), lambda i: (0, 0)),
        ],
        out_specs=pl.BlockSpec((EDGE_BLK, D), lambda i: (i, 0)),
        out_shape=jax.ShapeDtypeStruct((nrows, D), jnp.float32),
    )(comb, xj, Wd, bd, Wlc, bl)


def _node_mlp_body(p00_ref, p01_ref, p10_ref, p11_ref, x_ref, W1_ref, b1_ref,
                   W2_ref, b2_ref, out_ref, *, apply_relu):
    out = (p00_ref[0] + p01_ref[0] + p10_ref[0] + p11_ref[0] + x_ref[...])
    h = jnp.maximum(
        jnp.dot(out, W1_ref[...], preferred_element_type=jnp.float32)
        + b1_ref[...], 0.0)
    h = jnp.dot(h, W2_ref[...], preferred_element_type=jnp.float32) + b2_ref[...]
    if apply_relu:
        h = jnp.maximum(h, 0.0)
    out_ref[...] = h


def _node_mlp(parts0, parts1, x, W1, b1, W2, b2, apply_relu):
    grid = (N // NODE_BLK,)
    return pl.pallas_call(
        functools.partial(_node_mlp_body, apply_relu=apply_relu),
        grid=grid,
        in_specs=[
            pl.BlockSpec((1, NODE_BLK, D), lambda i: (0, i, 0)),
            pl.BlockSpec((1, NODE_BLK, D), lambda i: (1, i, 0)),
            pl.BlockSpec((1, NODE_BLK, D), lambda i: (0, i, 0)),
            pl.BlockSpec((1, NODE_BLK, D), lambda i: (1, i, 0)),
            pl.BlockSpec((NODE_BLK, D), lambda i: (i, 0)),
            pl.BlockSpec((D, D), lambda i: (0, 0)),
            pl.BlockSpec((1, D), lambda i: (0, 0)),
            pl.BlockSpec((D, D), lambda i: (0, 0)),
            pl.BlockSpec((1, D), lambda i: (0, 0)),
        ],
        out_specs=pl.BlockSpec((NODE_BLK, D), lambda i: (i, 0)),
        out_shape=jax.ShapeDtypeStruct((N, D), jnp.float32),
    )(parts0, parts0, parts1, parts1, x, W1, b1, W2, b2)


def kernel(x, edge_weight, edge_attr, We0, Wd0, bd0, Wl0, bl0, W1_0, b1_0,
           W2_0, b2_0, Wd1, bd1, Wl1, bl1, W1_1, b1_1, W2_1, b2_1, edge_index):
    ei2d = edge_index.reshape(2, NCHUNK, CHUNK)
    src2d = ei2d[0]
    dst2d = ei2d[1]
    cut_g, t_g = _edge_scalars(edge_weight.reshape(NCHUNK, CHUNK))
    comb = jnp.concatenate(
        [edge_attr, cut_g.reshape(E, 1), t_g.reshape(E, 1)], axis=1)
    Wlc0 = We0 @ Wl0
    Wlc1 = We0 @ Wl1
    zrows = jnp.zeros((ROWS_PER_SUB, D), jnp.float32)

    HCH = NCHUNK // 2
    HBLK = (HCH * CHUNK) // EDGE_BLK

    def layer(h, Wd, bd, Wlc, bl, W1, b1, W2, b2, apply_relu):
        xj0 = _sc_gather(h, src2d, 0, HCH)
        xj1 = _sc_gather(h, src2d, HCH, HCH)
        msg0 = _edge_messages(comb, xj0, Wd, bd[None, :], Wlc, bl[None, :],
                              0, HBLK)
        msg1 = _edge_messages(comb, xj1, Wd, bd[None, :], Wlc, bl[None, :],
                              HBLK, HBLK)
        parts0 = _sc_scatter_add(msg0, dst2d, zrows, 0, HCH)
        parts1 = _sc_scatter_add(msg1, dst2d, zrows, HCH, HCH)
        return _node_mlp(parts0, parts1, h, W1, b1[None, :], W2, b2[None, :],
                         apply_relu)

    h = layer(x, Wd0, bd0, Wlc0, bl0, W1_0, b1_0, W2_0, b2_0, True)
    h = layer(h, Wd1, bd1, Wlc1, bl1, W1_1, b1_1, W2_1, b2_1, False)
    return h
